# baseline (speedup 1.0000x reference)
import functools

import jax
import jax.numpy as jnp
from jax import lax
from jax.experimental import pallas as pl
from jax.experimental.pallas import tpu as pltpu
from jax.experimental.pallas import tpu_sc as plsc

_NC = 2
_NS = 16
_NW = _NC * _NS
_LANES = 128
_VL = 16
_NB = 2


@functools.cache
def _make_lookup(batch: int, hist: int, vocab: int, dim: int):
  n_blocks = batch // (_NW * _NB)
  assert n_blocks * _NW * _NB == batch and n_blocks % 2 == 0
  n_pairs = n_blocks // 2
  assert n_pairs >= 3
  hp = 56
  per_tile = n_blocks * _NB * hp

  mesh = plsc.VectorSubcoreMesh(core_axis_name="c", subcore_axis_name="s")

  @functools.partial(
      pl.kernel,
      mesh=mesh,
      out_type=jax.ShapeDtypeStruct((batch, hist, dim), jnp.float32),
      scratch_types=[
          pltpu.VMEM((per_tile,), jnp.int32),
          pltpu.VMEM((_NB, hist, _LANES), jnp.float32),
          pltpu.VMEM((_NB, hist, _LANES), jnp.float32),
          pltpu.VMEM((_NB, hist, dim), jnp.float32),
          pltpu.VMEM((_NB, hist, dim), jnp.float32),
          pltpu.SemaphoreType.DMA,
          pltpu.SemaphoreType.DMA,
          pltpu.SemaphoreType.DMA,
      ],
  )
  def body(
      idx_hbm,
      table_hbm,
      out_hbm,
      idx_v,
      rows0,
      rows1,
      comp0,
      comp1,
      sem_g,
      sem_o0,
      sem_o1,
  ):
    wid = lax.axis_index("s") * _NC + lax.axis_index("c")
    base = wid * n_blocks * _NB

    def fire_gathers(rows, j):
      for r in range(_NB):
        off = pl.multiple_of(j * (_NB * hp) + r * hp, 8)
        pltpu.async_copy(
            table_hbm.at[idx_v.at[pl.ds(off, hist)]], rows.at[r], sem_g
        )

    def drain_gathers(rows):
      for r in range(_NB):
        pltpu.make_async_copy(
            table_hbm.at[idx_v.at[pl.ds(0, hist)]], rows.at[r], sem_g
        ).wait()

    def repack(rows, comp):
      def per_r(r, _):
        for h in range(hist):
          for v in range(dim // _VL):
            comp[r, h, pl.ds(v * _VL, _VL)] = rows[r, h, pl.ds(v * _VL, _VL)]
        return 0

      lax.fori_loop(0, _NB, per_r, 0, unroll=False)

    def fire_write(j, comp, sem):
      off = base + j * _NB
      pltpu.async_copy(comp, out_hbm.at[pl.ds(off, _NB)], sem)

    def drain_write(comp, sem):
      pltpu.make_async_copy(comp, out_hbm.at[pl.ds(0, _NB)], sem).wait()

    idx_off = pl.multiple_of(wid * per_tile, 8)
    pltpu.sync_copy(idx_hbm.at[pl.ds(idx_off, per_tile)], idx_v)

    fire_gathers(rows0, 0)
    drain_gathers(rows0)
    fire_gathers(rows1, 1)
    repack(rows0, comp0)
    fire_write(0, comp0, sem_o0)
    drain_gathers(rows1)
    fire_gathers(rows0, 2)
    repack(rows1, comp1)
    fire_write(1, comp1, sem_o1)
    drain_gathers(rows0)
    fire_gathers(rows1, 3)
    drain_write(comp0, sem_o0)
    repack(rows0, comp0)
    fire_write(2, comp0, sem_o0)

    def step(p, _):
      j = 2 * p
      drain_gathers(rows1)
      fire_gathers(rows0, j)
      drain_write(comp1, sem_o1)
      repack(rows1, comp1)
      fire_write(j - 1, comp1, sem_o1)
      drain_gathers(rows0)
      fire_gathers(rows1, j + 1)
      drain_write(comp0, sem_o0)
      repack(rows0, comp0)
      fire_write(j, comp0, sem_o0)
      return 0

    lax.fori_loop(2, n_pairs - 1, step, 0, unroll=False)

    j = n_blocks - 2
    drain_gathers(rows1)
    fire_gathers(rows0, j)
    drain_write(comp1, sem_o1)
    repack(rows1, comp1)
    fire_write(j - 1, comp1, sem_o1)
    drain_gathers(rows0)
    fire_gathers(rows1, j + 1)
    drain_write(comp0, sem_o0)
    repack(rows0, comp0)
    fire_write(j, comp0, sem_o0)
    drain_gathers(rows1)
    drain_write(comp1, sem_o1)
    repack(rows1, comp1)
    fire_write(j + 1, comp1, sem_o1)
    drain_write(comp0, sem_o0)
    drain_write(comp1, sem_o1)

  return body


def kernel(input, lookup):
  batch, hist = input.shape
  vocab, dim = lookup.shape
  idx = jnp.pad(input, ((0, 0), (0, 56 - hist))).reshape(batch * 56)
  table = jnp.pad(lookup, ((0, 0), (0, _LANES - dim)))
  return _make_lookup(batch, hist, vocab, dim)(idx, table)
